# baseline (speedup 1.0000x reference)
import functools

import jax
import jax.numpy as jnp
from jax import lax
from jax.experimental import pallas as pl
from jax.experimental.pallas import tpu as pltpu
from jax.experimental.pallas import tpu_sc as plsc

N = 10000
E = 320000
D = 128
DEXT = 144

NC = 2
NS = 16
NW = NC * NS

CHUNK = 64
NBUF = 4
GRP = 16
KCH = 160
EPT = KCH * CHUNK
EPAD = EPT * NW

NTAB = N + 16
RPT = N // NS

BN = 2000


def _sc_aggregate(feat, src_t, dst_t, z_feat, fw):
    scratch = dict(
        src_v=pltpu.VMEM((GRP, CHUNK), jnp.int32),
        dst_v=pltpu.VMEM((GRP, CHUNK), jnp.int32),
        acc_sh=pltpu.VMEM_SHARED((N, fw), jnp.float32),
    )
    for k in range(NBUF):
        scratch[f"rows{k}_v"] = pltpu.VMEM((CHUNK, fw), jnp.float32)
        scratch[f"sg{k}"] = pltpu.SemaphoreType.DMA
        scratch[f"ss{k}"] = pltpu.SemaphoreType.DMA

    mesh = plsc.VectorSubcoreMesh(core_axis_name="c", subcore_axis_name="s")

    @functools.partial(
        pl.kernel,
        out_type=jax.ShapeDtypeStruct((NC, N, fw), jnp.float32),
        mesh=mesh, scratch_types=scratch,
        compiler_params=pltpu.CompilerParams(use_tc_tiling_on_sc=False))
    def run(feat_hbm, src_hbm, dst_hbm, zf_hbm, acc_out, *, src_v, dst_v,
            acc_sh, **bufsem):
        c = lax.axis_index("c")
        s = lax.axis_index("s")
        wid = s * NC + c
        base = s * RPT
        bufs = [bufsem[f"rows{k}_v"] for k in range(NBUF)]
        sgs = [bufsem[f"sg{k}"] for k in range(NBUF)]
        sss = [bufsem[f"ss{k}"] for k in range(NBUF)]

        pltpu.sync_copy(zf_hbm, acc_sh.at[pl.ds(base, RPT)])
        plsc.subcore_barrier()

        def body(g, carry):
            off = wid * KCH + g * GRP
            pltpu.sync_copy(src_hbm.at[pl.ds(off, GRP)], src_v)
            pltpu.sync_copy(dst_hbm.at[pl.ds(off, GRP)], dst_v)
            gcp = [None] * NBUF
            scp = [None] * NBUF
            for k in range(NBUF):
                gcp[k] = pltpu.async_copy(
                    feat_hbm.at[src_v.at[k]], bufs[k], sgs[k])
            for j in range(GRP):
                p = j % NBUF
                nj = j + NBUF - 2
                if j >= 2 and nj < GRP:
                    q = nj % NBUF
                    scp[q].wait()
                    scp[q] = None
                    gcp[q] = pltpu.async_copy(
                        feat_hbm.at[src_v.at[nj]], bufs[q], sgs[q])
                gcp[p].wait()
                scp[p] = pltpu.async_copy(
                    bufs[p], acc_sh.at[dst_v.at[j]], sss[p], add=True)
            for k in range(NBUF):
                if scp[k] is not None:
                    scp[k].wait()
            return carry

        lax.fori_loop(0, KCH // GRP, body, 0)
        plsc.subcore_barrier()

        pltpu.sync_copy(acc_sh.at[pl.ds(base, RPT)],
                        acc_out.at[c].at[pl.ds(base, RPT)])

    return run(feat, src_t, dst_t, z_feat)


def _tc_layer1_kernel(x_ref, a0_ref, a1_ref, ws_ref, wn_ref, b_ref, out_ref):
    acc = a0_ref[...] + a1_ref[...]
    deg = acc[:, D:D + 1]
    inv = 1.0 / jnp.maximum(deg, 1.0)
    agg = acc[:, :D] * inv
    out = (jnp.dot(x_ref[...], ws_ref[...], preferred_element_type=jnp.float32)
           + jnp.dot(agg, wn_ref[...], preferred_element_type=jnp.float32)
           + b_ref[...])
    out_ref[...] = jnp.maximum(out, 0.0)


def _tc_layer2_kernel(h_ref, a0_ref, a1_ref, d0_ref, d1_ref,
                      ws_ref, wn_ref, b_ref, out_ref):
    deg = d0_ref[...] + d1_ref[...]
    inv = 1.0 / jnp.maximum(deg, 1.0)
    agg = (a0_ref[...] + a1_ref[...]) * inv
    out = (jnp.dot(h_ref[...], ws_ref[...], preferred_element_type=jnp.float32)
           + jnp.dot(agg, wn_ref[...], preferred_element_type=jnp.float32)
           + b_ref[...])
    out_ref[...] = jnp.maximum(out, 0.0)


def _row_spec(w):
    return pl.BlockSpec((BN, w), lambda i: (i, 0))


_W_SPEC = pl.BlockSpec((D, D), lambda i: (0, 0))
_B_SPEC = pl.BlockSpec((1, D), lambda i: (0, 0))


def _tc_layer1(x, acc, W_self, W_neigh, b):
    return pl.pallas_call(
        _tc_layer1_kernel,
        grid=(N // BN,),
        in_specs=[_row_spec(D), _row_spec(DEXT), _row_spec(DEXT),
                  _W_SPEC, _W_SPEC, _B_SPEC],
        out_specs=_row_spec(D),
        out_shape=jax.ShapeDtypeStruct((N, D), jnp.float32),
    )(x, acc[0], acc[1], W_self, W_neigh, b.reshape(1, D))


def _tc_layer2(h, acc, deg0, deg1, W_self, W_neigh, b):
    return pl.pallas_call(
        _tc_layer2_kernel,
        grid=(N // BN,),
        in_specs=[_row_spec(D), _row_spec(D), _row_spec(D),
                  _row_spec(1), _row_spec(1), _W_SPEC, _W_SPEC, _B_SPEC],
        out_specs=_row_spec(D),
        out_shape=jax.ShapeDtypeStruct((N, D), jnp.float32),
    )(h, acc[0], acc[1], deg0, deg1, W_self, W_neigh, b.reshape(1, D))


def kernel(x, edge_index, W_self1, W_neigh1, b1, W_self2, W_neigh2, b2):
    src = edge_index[0]
    dst = edge_index[1]
    pad_e = EPAD - E
    src_t = jnp.concatenate(
        [src, jnp.full((pad_e,), N, jnp.int32)]).reshape(NW * KCH, CHUNK)
    dst_t = jnp.concatenate(
        [dst, jnp.zeros((pad_e,), jnp.int32)]).reshape(NW * KCH, CHUNK)

    xe = jnp.zeros((NTAB, DEXT), jnp.float32)
    xe = xe.at[:N, :D].set(x).at[:N, D].set(1.0)

    acc1 = _sc_aggregate(xe, src_t, dst_t,
                         jnp.zeros((RPT, DEXT), jnp.float32), DEXT)
    h1 = _tc_layer1(x, acc1, W_self1, W_neigh1, b1)

    h1p = jnp.zeros((NTAB, D), jnp.float32).at[:N].set(h1)
    acc2 = _sc_aggregate(h1p, src_t, dst_t,
                         jnp.zeros((RPT, D), jnp.float32), D)
    deg0 = acc1[0, :, D:D + 1]
    deg1 = acc1[1, :, D:D + 1]
    return _tc_layer2(h1, acc2, deg0, deg1, W_self2, W_neigh2, b2)


# speedup vs baseline: 1.3644x; 1.3644x over previous
import functools

import jax
import jax.numpy as jnp
from jax import lax
from jax.experimental import pallas as pl
from jax.experimental.pallas import tpu as pltpu
from jax.experimental.pallas import tpu_sc as plsc

N = 10000
E = 320000
D = 128
DEG_W = 16

NC = 2
NS = 16
NW = NC * NS

CHUNK = 128
GRP = 16
KCH = 80
EPT = KCH * CHUNK
EPAD = EPT * NW

NACC = N + 16
RPT = N // NS

BN = 2000


def _interleave_bf16(a):
    n = a.shape[0]
    b = a.astype(jnp.bfloat16).reshape(n, D // 32, 2, 16)
    return b.transpose(0, 1, 3, 2).reshape(n, D)


def _sc_aggregate(feat_bf, src_t, dst_t, z_feat, z_deg, ones_in, with_deg):
    out_type = [jax.ShapeDtypeStruct((NC, N, D), jnp.float32)]
    if with_deg:
        out_type.append(jax.ShapeDtypeStruct((NC, N, DEG_W), jnp.float32))

    scratch = dict(
        src_v=pltpu.VMEM((GRP, CHUNK), jnp.int32),
        dst_v=pltpu.VMEM((GRP, CHUNK), jnp.int32),
        bf0_v=pltpu.VMEM((CHUNK, D), jnp.bfloat16),
        bf1_v=pltpu.VMEM((CHUNK, D), jnp.bfloat16),
        fr_v=pltpu.VMEM((CHUNK, D), jnp.float32),
        sg0=pltpu.SemaphoreType.DMA,
        sg1=pltpu.SemaphoreType.DMA,
        ssf=pltpu.SemaphoreType.DMA,
        acc_sh=pltpu.VMEM_SHARED((NACC, D), jnp.float32),
    )
    if with_deg:
        scratch.update(
            ones_v=pltpu.VMEM((CHUNK, DEG_W), jnp.float32),
            ssd=pltpu.SemaphoreType.DMA,
            deg_sh=pltpu.VMEM_SHARED((NACC, DEG_W), jnp.float32),
        )

    mesh = plsc.VectorSubcoreMesh(core_axis_name="c", subcore_axis_name="s")

    @functools.partial(
        pl.kernel, out_type=tuple(out_type), mesh=mesh, scratch_types=scratch,
        compiler_params=pltpu.CompilerParams(use_tc_tiling_on_sc=False,
                                             needs_layout_passes=False))
    def run(feat_hbm, src_hbm, dst_hbm, zf_hbm, zd_hbm, ones_hbm, *outs,
            src_v, dst_v, bf0_v, bf1_v, fr_v, sg0, sg1, ssf, acc_sh,
            ones_v=None, ssd=None, deg_sh=None):
        if with_deg:
            acc_out, deg_out = outs
        else:
            (acc_out,) = outs
        c = lax.axis_index("c")
        s = lax.axis_index("s")
        wid = s * NC + c
        base = s * RPT
        bufs = (bf0_v, bf1_v)
        sgs = (sg0, sg1)

        pltpu.sync_copy(zf_hbm, acc_sh.at[pl.ds(base, RPT)])
        if with_deg:
            pltpu.sync_copy(zd_hbm, deg_sh.at[pl.ds(base, RPT)])
            pltpu.sync_copy(ones_hbm, ones_v)
        plsc.subcore_barrier()

        def convert(p):
            bfp = bufs[p]

            def conv_body(r, carry):
                for u in range(2):
                    for k in range(D // 32):
                        v = bfp[r * 2 + u, pl.ds(32 * k, 32)]
                        a, b = plsc.unpack(
                            v, format=plsc.PackFormat.INTERLEAVED)
                        fr_v[r * 2 + u, pl.ds(32 * k, 16)] = a
                        fr_v[r * 2 + u, pl.ds(32 * k + 16, 16)] = b
                return carry

            lax.fori_loop(0, CHUNK // 2, conv_body, 0)

        def body(g, carry):
            off = wid * KCH + g * GRP
            pltpu.sync_copy(src_hbm.at[pl.ds(off, GRP)], src_v)
            pltpu.sync_copy(dst_hbm.at[pl.ds(off, GRP)], dst_v)
            gcp = [
                pltpu.async_copy(feat_hbm.at[src_v.at[0]], bufs[0], sgs[0]),
                pltpu.async_copy(feat_hbm.at[src_v.at[1]], bufs[1], sgs[1]),
            ]
            scp = None
            sdp = None
            for j in range(GRP):
                p = j % 2
                gcp[p].wait()
                if scp is not None:
                    scp.wait()
                convert(p)
                if j + 2 < GRP:
                    gcp[p] = pltpu.async_copy(
                        feat_hbm.at[src_v.at[j + 2]], bufs[p], sgs[p])
                scp = pltpu.async_copy(
                    fr_v, acc_sh.at[dst_v.at[j]], ssf, add=True)
                if with_deg:
                    if sdp is not None:
                        sdp.wait()
                    sdp = pltpu.async_copy(
                        ones_v, deg_sh.at[dst_v.at[j]], ssd, add=True)
            scp.wait()
            if with_deg:
                sdp.wait()
            return carry

        lax.fori_loop(0, KCH // GRP, body, 0)
        plsc.subcore_barrier()

        pltpu.sync_copy(acc_sh.at[pl.ds(base, RPT)],
                        acc_out.at[c].at[pl.ds(base, RPT)])
        if with_deg:
            pltpu.sync_copy(deg_sh.at[pl.ds(base, RPT)],
                            deg_out.at[c].at[pl.ds(base, RPT)])

    return run(feat_bf, src_t, dst_t, z_feat, z_deg, ones_in)


def _tc_layer_kernel(h_ref, a0_ref, a1_ref, d0_ref, d1_ref,
                     ws_ref, wn_ref, b_ref, out_ref):
    deg = (d0_ref[...] + d1_ref[...])[:, 0:1]
    inv = 1.0 / jnp.maximum(deg, 1.0)
    agg = (a0_ref[...] + a1_ref[...]) * inv
    out = (jnp.dot(h_ref[...], ws_ref[...], preferred_element_type=jnp.float32)
           + jnp.dot(agg, wn_ref[...], preferred_element_type=jnp.float32)
           + b_ref[...])
    out_ref[...] = jnp.maximum(out, 0.0)


def _row_spec(w):
    return pl.BlockSpec((BN, w), lambda i: (i, 0))


_W_SPEC = pl.BlockSpec((D, D), lambda i: (0, 0))
_B_SPEC = pl.BlockSpec((1, D), lambda i: (0, 0))


def _tc_layer(h, acc, deg, W_self, W_neigh, b):
    return pl.pallas_call(
        _tc_layer_kernel,
        grid=(N // BN,),
        in_specs=[_row_spec(D), _row_spec(D), _row_spec(D),
                  _row_spec(DEG_W), _row_spec(DEG_W),
                  _W_SPEC, _W_SPEC, _B_SPEC],
        out_specs=_row_spec(D),
        out_shape=jax.ShapeDtypeStruct((N, D), jnp.float32),
    )(h, acc[0], acc[1], deg[0], deg[1], W_self, W_neigh, b.reshape(1, D))


def kernel(x, edge_index, W_self1, W_neigh1, b1, W_self2, W_neigh2, b2):
    src = edge_index[0]
    dst = edge_index[1]
    pad_e = EPAD - E
    src_t = jnp.concatenate(
        [src, jnp.zeros((pad_e,), jnp.int32)]).reshape(NW * KCH, CHUNK)
    dst_t = jnp.concatenate(
        [dst, jnp.full((pad_e,), N, jnp.int32)]).reshape(NW * KCH, CHUNK)

    z_feat = jnp.zeros((RPT, D), jnp.float32)
    z_deg = jnp.zeros((RPT, DEG_W), jnp.float32)
    ones_in = jnp.ones((CHUNK, DEG_W), jnp.float32)

    xb = _interleave_bf16(x)
    acc1, deg = _sc_aggregate(xb, src_t, dst_t, z_feat, z_deg, ones_in,
                              with_deg=True)
    h1 = _tc_layer(x, acc1, deg, W_self1, W_neigh1, b1)

    h1b = _interleave_bf16(h1)
    (acc2,) = _sc_aggregate(h1b, src_t, dst_t, z_feat, z_deg, ones_in,
                            with_deg=False)
    return _tc_layer(h1, acc2, deg, W_self2, W_neigh2, b2)
